# Initial kernel scaffold; baseline (speedup 1.0000x reference)
import jax
import jax.numpy as jnp
from jax.experimental import pallas as pl


def kernel(idx, targets, table):
    raise NotImplementedError("write your pallas kernel here")


# speedup vs baseline: 1.5582x; 1.5582x over previous
import functools

import jax
import jax.numpy as jnp
from jax import lax
from jax.experimental import pallas as pl
from jax.experimental.pallas import tpu as pltpu
from jax.experimental.pallas import tpu_sc as plsc

V = 1000
NTOK = 1024 * 50
NC, NS = 2, 16
NW = NC * NS
PER_TILE = NTOK // NW
CHUNK = 32
NCHUNK = PER_TILE // CHUNK
GRP = CHUNK // 16


def _lse_body(tab_ref, lse_ref):
    x = tab_ref[...]
    m = jnp.max(x, axis=1)
    s = jnp.sum(jnp.exp(x - m[:, None]), axis=1)
    lse_ref[...] = m + jnp.log(s)


def _compute_lse(table):
    return pl.pallas_call(
        _lse_body,
        out_shape=jax.ShapeDtypeStruct((V,), jnp.float32),
    )(table)


_sc_mesh = plsc.VectorSubcoreMesh(core_axis_name="c", subcore_axis_name="s")


@functools.partial(
    pl.kernel,
    out_type=[
        jax.ShapeDtypeStruct((NTOK, V), jnp.float32),
        jax.ShapeDtypeStruct((NW, 16), jnp.float32),
    ],
    mesh=_sc_mesh,
    compiler_params=pltpu.CompilerParams(
        use_tc_tiling_on_sc=False, needs_layout_passes=False),
    scratch_types=[
        [pltpu.VMEM((CHUNK,), jnp.int32)] * 2,
        [pltpu.VMEM((CHUNK,), jnp.int32)] * 2,
        [pltpu.VMEM((CHUNK, V), jnp.float32)] * 2,
        pltpu.VMEM((V,), jnp.float32),
        pltpu.VMEM((16,), jnp.float32),
        [pltpu.SemaphoreType.DMA] * 2,
        [pltpu.SemaphoreType.DMA] * 2,
    ],
)
def _sc_gather(idx_hbm, tgt_hbm, lse_hbm, table_hbm, out_hbm, part_hbm,
               idx_v, tgt_v, rows_v, lse_v, acc_v, gsem, ssem):
    cid = lax.axis_index("c")
    sid = lax.axis_index("s")
    wid = sid * NC + cid
    base = wid * PER_TILE

    pltpu.sync_copy(lse_hbm, lse_v)

    def prep(c, s):
        off = base + c * CHUNK
        pltpu.sync_copy(idx_hbm.at[pl.ds(off, CHUNK)], idx_v[s])
        pltpu.sync_copy(tgt_hbm.at[pl.ds(off, CHUNK)], tgt_v[s])
        pltpu.async_copy(table_hbm.at[idx_v[s]], rows_v[s], gsem[s])

    def wait_gather(s):
        pltpu.make_async_copy(table_hbm.at[idx_v[s]], rows_v[s], gsem[s]).wait()

    def wait_scatter(c, s):
        off = base + c * CHUNK
        pltpu.make_async_copy(
            rows_v[s], out_hbm.at[pl.ds(off, CHUNK)], ssem[s]).wait()

    def compute(acc, s):
        for j in range(GRP):
            rid = lax.iota(jnp.int32, 16) + j * 16
            tg = tgt_v[s][pl.ds(j * 16, 16)]
            ii = idx_v[s][pl.ds(j * 16, 16)]
            tv = plsc.load_gather(rows_v[s], [rid, tg])
            lv = plsc.load_gather(lse_v, [ii])
            acc = acc + (lv - tv)
        return acc

    prep(0, 0)
    prep(1, 1)

    @pl.loop(0, NCHUNK - 2, step=2, init_carry=jnp.zeros((16,), jnp.float32))
    def acc_loop(g, acc):
        for s in range(2):
            c = g + s
            wait_gather(s)
            acc = compute(acc, s)
            off = base + c * CHUNK
            pltpu.async_copy(rows_v[s], out_hbm.at[pl.ds(off, CHUNK)], ssem[s])
            offn = base + (c + 2) * CHUNK
            pltpu.sync_copy(idx_hbm.at[pl.ds(offn, CHUNK)], idx_v[s])
            pltpu.sync_copy(tgt_hbm.at[pl.ds(offn, CHUNK)], tgt_v[s])
            wait_scatter(c, s)
            pltpu.async_copy(table_hbm.at[idx_v[s]], rows_v[s], gsem[s])
        return acc

    acc = acc_loop
    for s in range(2):
        c = NCHUNK - 2 + s
        wait_gather(s)
        acc = compute(acc, s)
        off = base + c * CHUNK
        pltpu.sync_copy(rows_v[s], out_hbm.at[pl.ds(off, CHUNK)])

    acc_v[...] = acc
    pltpu.sync_copy(acc_v, part_hbm.at[wid])


def kernel(idx, targets, table):
    idx_f = idx.reshape(-1).astype(jnp.int32)
    tgt_f = targets.reshape(-1).astype(jnp.int32)
    lse = _compute_lse(table)
    logits_flat, partials = _sc_gather(idx_f, tgt_f, lse, table)
    loss = jnp.sum(partials) / jnp.float32(NTOK)
    return logits_flat.reshape(idx.shape + (V,)), loss
